# baseline (speedup 1.0000x reference)
import functools

import jax
import jax.numpy as jnp
from jax import lax
from jax.experimental import pallas as pl
from jax.experimental.pallas import tpu as pltpu
from jax.experimental.pallas import tpu_sc as plsc

N = 10000
NP = 10112
D = 128
K = 128
NC = 2
NS = 16
W = NC * NS
CH = 84
REC = K + K * 8
ROWS_PER_TILE = NP // NS


def _widen_idx(v):
    return jnp.where(v > 65535, lax.shift_right_logical(v, 16), v)


def _spmm_sc(table, combo, zeros):
    mesh = plsc.VectorSubcoreMesh(core_axis_name="c", subcore_axis_name="s")

    @functools.partial(
        pl.kernel,
        out_type=jax.ShapeDtypeStruct((NC * NP, D), jnp.float32),
        mesh=mesh,
        compiler_params=pltpu.CompilerParams(use_tc_tiling_on_sc=False,
                                             needs_layout_passes=False),
        scratch_types=[
            [pltpu.VMEM((K,), jnp.int32)] * 3,
            [pltpu.VMEM((REC,), jnp.float32)] * 6,
            [pltpu.VMEM((K,), jnp.int32)] * 3,
            [pltpu.VMEM((K, D // 2), jnp.int32)] * 3,
            pltpu.VMEM((K, D), jnp.float32),
            pltpu.VMEM_SHARED((NP, D), jnp.float32),
            [pltpu.SemaphoreType.DMA] * 3,
            [pltpu.SemaphoreType.DMA] * 6,
            [pltpu.SemaphoreType.DMA] * 3,
        ],
    )
    def spmm(table_h, combo_h, zeros_h, out_h,
             colbuf, rec, rowbuf, gbuf, sbuf, acc, gsem, csem, ssem):
        cid = lax.axis_index("c")
        sid = lax.axis_index("s")
        wid = sid * NC + cid
        r0 = sid * ROWS_PER_TILE
        pltpu.sync_copy(zeros_h.at[pl.ds(r0, ROWS_PER_TILE)],
                        acc.at[pl.ds(r0, ROWS_PER_TILE)])
        plsc.subcore_barrier()

        def fetch_rec(i, r):
            pltpu.async_copy(
                combo_h.at[pl.ds((wid * CH + i) * REC, REC)], rec[r], csem[r])

        def wait_rec(r):
            pltpu.make_async_copy(
                combo_h.at[pl.ds(0, REC)], rec[r], csem[r]).wait()

        def unpack_idx(r, off, dst):
            for g in range(K // 32):
                pair = plsc.bitcast(rec[r][pl.ds(off + g * 16, 16)],
                                    jnp.uint16)
                a, b = plsc.unpack(pair,
                                   format=plsc.PackFormat.INTERLEAVED,
                                   preferred_element_type=jnp.int32)
                dst[pl.ds(g * 32, 16)] = _widen_idx(a)
                dst[pl.ds(g * 32 + 16, 16)] = _widen_idx(b)

        def start_gather(i, r, s):
            unpack_idx(r, K // 2, colbuf[s])
            pltpu.async_copy(table_h.at[colbuf[s]], gbuf[s], gsem[s])

        def wait_gather(s):
            pltpu.make_async_copy(
                table_h.at[pl.ds(0, K)], gbuf[s], gsem[s]).wait()

        def wait_scatter(s):
            pltpu.make_async_copy(
                sbuf, acc.at[pl.ds(0, K)], ssem[s]).wait()

        for j in range(4):
            fetch_rec(j, j)
        wait_rec(0)
        start_gather(0, 0, 0)
        wait_rec(1)
        start_gather(1, 1, 1)

        def six(t, carry):
            for q in range(6):
                i = 6 * t + q
                r = q
                s = q % 3
                s2 = (q + 2) % 3
                r2 = (q + 2) % 6
                r4 = (q + 4) % 6

                @pl.when(i >= 2)
                def _():
                    wait_rec(r)

                unpack_idx(r, 0, rowbuf[s])

                @pl.when(i >= 1)
                def _():
                    wait_scatter(s2)

                @pl.when(i + 2 < CH)
                def _():
                    start_gather(i + 2, r2, s2)

                @pl.when(i + 4 < CH)
                def _():
                    fetch_rec(i + 4, r4)

                wait_gather(s)

                def edge_pair(j, c2):
                    vp = plsc.bitcast(
                        rec[r][pl.ds(K + j * 16, 16)], jnp.bfloat16)
                    va, vb = plsc.unpack(
                        vp, format=plsc.PackFormat.INTERLEAVED,
                        preferred_element_type=jnp.float32)
                    for d, splat in ((0, va), (1, vb)):
                        k = 2 * j + d
                        for u in range(D // 32):
                            w = gbuf[s][k, pl.ds(u * 16, 16)]
                            p2 = plsc.bitcast(w, jnp.bfloat16)
                            a, b = plsc.unpack(
                                p2, format=plsc.PackFormat.INTERLEAVED,
                                preferred_element_type=jnp.float32)
                            sbuf[k, pl.ds(u * 32, 16)] = a * splat
                            sbuf[k, pl.ds(u * 32 + 16, 16)] = b * splat
                    return c2

                lax.fori_loop(0, K // 2, edge_pair, 0)
                pltpu.async_copy(sbuf, acc.at[rowbuf[s]], ssem[s],
                                 add=True)
            return carry

        lax.fori_loop(0, CH // 6, six, 0)
        wait_scatter((CH - 1) % 3)
        plsc.subcore_barrier()
        pltpu.sync_copy(acc.at[pl.ds(r0, ROWS_PER_TILE)],
                        out_h.at[pl.ds(cid * NP + r0, ROWS_PER_TILE)])

    return spmm(table, combo, zeros)


_BN = 2000


def _tanh_combine(p):
    def body(p0_ref, p1_ref, o_ref):
        o_ref[...] = jnp.tanh(p0_ref[...] + p1_ref[...])

    return pl.pallas_call(
        body,
        grid=(N // _BN,),
        in_specs=[pl.BlockSpec((_BN, D), lambda i: (i, 0)),
                  pl.BlockSpec((_BN, D), lambda i: (i, 0))],
        out_specs=pl.BlockSpec((_BN, D), lambda i: (i, 0)),
        out_shape=jax.ShapeDtypeStruct((N, D), jnp.float32),
    )(p[:N], p[NP:NP + N])


def _final_sum(x, t1, p):
    def body(x_ref, t1_ref, p0_ref, p1_ref, o_ref):
        o_ref[...] = (x_ref[...] + t1_ref[...]
                      + jnp.tanh(p0_ref[...] + p1_ref[...]))

    return pl.pallas_call(
        body,
        grid=(N // _BN,),
        in_specs=[pl.BlockSpec((_BN, D), lambda i: (i, 0))] * 4,
        out_specs=pl.BlockSpec((_BN, D), lambda i: (i, 0)),
        out_shape=jax.ShapeDtypeStruct((N, D), jnp.float32),
    )(x, t1, p[:N], p[NP:NP + N])


def _interleave_pairs(a):
    s = a.shape[:-1]
    n = a.shape[-1]
    return (a.reshape(s + (n // 32, 2, 16))
            .swapaxes(-1, -2).reshape(s + (n,)))


def _u16_words(a16):
    m, n = a16.shape
    return jax.lax.bitcast_convert_type(
        jax.lax.bitcast_convert_type(a16.reshape(m, n // 2, 2), jnp.int32),
        jnp.float32)


def kernel(inputs_weight, support_indices, support_values):
    x = inputs_weight[1:]
    rows = support_indices[0]
    cols = support_indices[1]
    vals = support_values
    e = vals.shape[0]
    e_pad = W * CH * K
    pad = e_pad - e
    cols_p = jnp.pad(cols, (0, pad))
    rows_p = jnp.pad(rows, (0, pad))
    vals_p = jnp.pad(vals, (0, pad))
    nch = W * CH

    rows_w = _u16_words(_interleave_pairs(
        rows_p.astype(jnp.uint16)).reshape(nch, K))
    cols_w = _u16_words(_interleave_pairs(
        cols_p.astype(jnp.uint16)).reshape(nch, K))
    vals_i = jnp.broadcast_to(
        vals_p.reshape(nch, K // 2, 1, 2).astype(jnp.bfloat16),
        (nch, K // 2, 16, 2))
    vals_w = jax.lax.bitcast_convert_type(
        jax.lax.bitcast_convert_type(vals_i, jnp.int32),
        jnp.float32).reshape(nch, K * 8)
    combo = jnp.concatenate([rows_w, cols_w, vals_w], axis=1).reshape(-1)
    zeros = jnp.zeros((NP, D), jnp.float32)

    def pack_table(t):
        ts = _interleave_pairs(t).astype(jnp.bfloat16).reshape(N, D // 2, 2)
        return jax.lax.bitcast_convert_type(ts, jnp.int32)

    p1 = _spmm_sc(pack_table(x), combo, zeros)
    t1 = _tanh_combine(p1)
    p2 = _spmm_sc(pack_table(t1), combo, zeros)
    out = _final_sum(x, t1, p2)
    return jnp.concatenate([inputs_weight[0:1], out], axis=0)
